# baseline (speedup 1.0000x reference)
import jax
import jax.numpy as jnp
from jax.experimental import pallas as pl


def _add_block(x_ref, pos_ref, out_ref):
    out_ref[...] = x_ref[...] + pos_ref[...][None, :, :]


def kernel(x, position_embedding):
    batch, seq_len, d = x.shape
    s_blk = 128
    grid = (seq_len // s_blk,)
    return pl.pallas_call(
        _add_block,
        grid=grid,
        in_specs=[
            pl.BlockSpec((batch, s_blk, d), lambda i: (0, i, 0)),
            pl.BlockSpec((s_blk, d), lambda i: (i, 0)),
        ],
        out_specs=pl.BlockSpec((batch, s_blk, d), lambda i: (0, i, 0)),
        out_shape=jax.ShapeDtypeStruct((batch, seq_len, d), x.dtype),
    )(x, position_embedding[:seq_len])


# speedup vs baseline: 1.0771x; 1.0771x over previous
import jax
import jax.numpy as jnp
from jax.experimental import pallas as pl


def _add_block(x_ref, pos_ref, out_ref):
    out_ref[...] = x_ref[...] + pos_ref[...][None, :, :]


def kernel(x, position_embedding):
    batch, seq_len, d = x.shape
    s_blk = 512
    grid = (seq_len // s_blk,)
    return pl.pallas_call(
        _add_block,
        grid=grid,
        in_specs=[
            pl.BlockSpec((batch, s_blk, d), lambda i: (0, i, 0)),
            pl.BlockSpec((s_blk, d), lambda i: (i, 0)),
        ],
        out_specs=pl.BlockSpec((batch, s_blk, d), lambda i: (0, i, 0)),
        out_shape=jax.ShapeDtypeStruct((batch, seq_len, d), x.dtype),
    )(x, position_embedding[:seq_len])
